# baseline (speedup 1.0000x reference)
import jax
import jax.numpy as jnp
from jax.experimental import pallas as pl
from jax.experimental.pallas import tpu as pltpu


def _cheby_kernel(x_ref, adj_a_ref, adj_b_ref, w_ref, b_ref, o_ref, s_ref):
    r = pl.program_id(0)
    ik = pl.program_id(1)

    @pl.when(r == 0)
    def _compute_support():
        w_i = w_ref[pl.ds(ik, 1), :, :][0]
        s = jnp.dot(x_ref[...], w_i, preferred_element_type=jnp.float32)
        s_ref[pl.ds(ik, 1), :, :] = s[None]

    @pl.when(ik == 0)
    def _init_out():
        o_ref[...] = jnp.broadcast_to(b_ref[...], o_ref.shape)

    s_i = s_ref[pl.ds(ik, 1), :, :][0]
    br = adj_a_ref.shape[1]
    o_ref[:br] += jnp.dot(adj_a_ref[0], s_i, preferred_element_type=jnp.float32)
    o_ref[br:] += jnp.dot(adj_b_ref[0], s_i, preferred_element_type=jnp.float32)


def kernel(input, adj, weight, bias):
    n, in_f = input.shape
    deg = adj.shape[0]
    out_f = weight.shape[-1]

    br = 512
    grid = (n // (2 * br), deg)

    out = pl.pallas_call(
        _cheby_kernel,
        grid=grid,
        in_specs=[
            pl.BlockSpec((n, in_f), lambda r, ik: (0, 0)),
            pl.BlockSpec((1, br, n), lambda r, ik: (ik, 2 * r, 0)),
            pl.BlockSpec((1, br, n), lambda r, ik: (ik, 2 * r + 1, 0)),
            pl.BlockSpec((deg, in_f, out_f), lambda r, ik: (0, 0, 0)),
            pl.BlockSpec((1, out_f), lambda r, ik: (0, 0)),
        ],
        out_specs=pl.BlockSpec((2 * br, out_f), lambda r, ik: (r, 0)),
        out_shape=jax.ShapeDtypeStruct((n, out_f), jnp.float32),
        scratch_shapes=[pltpu.VMEM((deg, n, out_f), jnp.float32)],
    )(input, adj, adj, weight, bias.reshape(1, out_f))
    return out


# speedup vs baseline: 1.0041x; 1.0041x over previous
import jax
import jax.numpy as jnp
from jax.experimental import pallas as pl
from jax.experimental.pallas import tpu as pltpu


def _cheby_kernel(x_ref, adj_ref, w_ref, b_ref, o_ref, s_ref, *, br):
    i = pl.program_id(0)
    r = pl.program_id(1)

    @pl.when(r == 0)
    def _compute_support():
        w_i = w_ref[pl.ds(i, 1), :, :][0]
        s_ref[...] = jnp.dot(x_ref[...], w_i, preferred_element_type=jnp.float32)

    @pl.when((i == 0) & (r == 0))
    def _init_out():
        o_ref[...] = jnp.broadcast_to(b_ref[...], o_ref.shape)

    o_ref[pl.ds(r * br, br), :] += jnp.dot(
        adj_ref[0], s_ref[...], preferred_element_type=jnp.float32
    )


def kernel(input, adj, weight, bias):
    n, in_f = input.shape
    deg = adj.shape[0]
    out_f = weight.shape[-1]

    br = 1024
    grid = (deg, n // br)

    import functools

    out = pl.pallas_call(
        functools.partial(_cheby_kernel, br=br),
        grid=grid,
        in_specs=[
            pl.BlockSpec((n, in_f), lambda i, r: (0, 0)),
            pl.BlockSpec((1, br, n), lambda i, r: (i, r, 0)),
            pl.BlockSpec((deg, in_f, out_f), lambda i, r: (0, 0, 0)),
            pl.BlockSpec((1, out_f), lambda i, r: (0, 0)),
        ],
        out_specs=pl.BlockSpec((n, out_f), lambda i, r: (0, 0)),
        out_shape=jax.ShapeDtypeStruct((n, out_f), jnp.float32),
        scratch_shapes=[pltpu.VMEM((n, out_f), jnp.float32)],
    )(input, adj, weight, bias.reshape(1, out_f))
    return out


# speedup vs baseline: 1.0301x; 1.0259x over previous
import jax
import jax.numpy as jnp
from jax.experimental import pallas as pl
from jax.experimental.pallas import tpu as pltpu


def _cheby_kernel(x_ref, adj_ref, w_ref, b_ref, o_ref, s_ref):
    r = pl.program_id(0)
    ik = pl.program_id(1)

    @pl.when(r == 0)
    def _compute_support():
        w_i = w_ref[pl.ds(ik, 1), :, :][0]
        s = jnp.dot(x_ref[...], w_i, preferred_element_type=jnp.float32)
        s_ref[pl.ds(ik, 1), :, :] = s[None]

    @pl.when(ik == 0)
    def _init_out():
        o_ref[...] = jnp.broadcast_to(b_ref[...], o_ref.shape)

    s_i = s_ref[pl.ds(ik, 1), :, :][0]
    o_ref[...] += jnp.dot(adj_ref[0], s_i, preferred_element_type=jnp.float32)


def kernel(input, adj, weight, bias):
    n, in_f = input.shape
    deg = adj.shape[0]
    out_f = weight.shape[-1]

    br = 512
    grid = (n // br, deg)

    out = pl.pallas_call(
        _cheby_kernel,
        grid=grid,
        in_specs=[
            pl.BlockSpec((n, in_f), lambda r, ik: (0, 0)),
            pl.BlockSpec((1, br, n), lambda r, ik: (ik, r, 0)),
            pl.BlockSpec((deg, in_f, out_f), lambda r, ik: (0, 0, 0)),
            pl.BlockSpec((1, out_f), lambda r, ik: (0, 0)),
        ],
        out_specs=pl.BlockSpec((br, out_f), lambda r, ik: (r, 0)),
        out_shape=jax.ShapeDtypeStruct((n, out_f), jnp.float32),
        scratch_shapes=[pltpu.VMEM((deg, n, out_f), jnp.float32)],
    )(input, adj, weight, bias.reshape(1, out_f))
    return out
